# baseline (speedup 1.0000x reference)
import functools

import jax
import jax.numpy as jnp
from jax.experimental import pallas as pl

B, C, D, H, W = 4, 128, 2, 200, 176
HW = H * W
PC = HW // 128
NSLOT = B * D * HW


def _tc_body(a, s, out_ref):
    t = a[...].T
    valid = (s[...] >= 0).reshape(1, 128)
    out_ref[0] = jnp.where(valid, t, 0.0)


@jax.jit
def _tc_transpose(a, sv3):
    return pl.pallas_call(
        _tc_body,
        grid=(B, D, PC),
        in_specs=[
            pl.BlockSpec((128, 128), lambda b, d, p: (b * 2 * PC + d * PC + p, 0)),
            pl.BlockSpec((1, 1, 128), lambda b, d, p: (b * 2 * PC + d * PC + p, 0, 0)),
        ],
        out_specs=pl.BlockSpec((1, 128, 128), lambda b, d, p: (b, 0, d * PC + p)),
        out_shape=jax.ShapeDtypeStruct((B, 128, D * HW), jnp.float32),
    )(a, sv3)


def kernel(voxel_features, voxel_lin_idx):
    lin = voxel_lin_idx.astype(jnp.int32)
    dense = jnp.zeros((NSLOT, C), dtype=voxel_features.dtype)
    dense = dense.at[lin].set(voxel_features)
    sv3 = jnp.zeros((B * D * PC, 1, 128), dtype=jnp.int32)
    out = _tc_transpose(dense, sv3)
    return out.reshape(B, C * D, H, W)


# speedup vs baseline: 1.8709x; 1.8709x over previous
import functools

import jax
import jax.numpy as jnp
from jax.experimental import pallas as pl

B, C, D, H, W = 4, 128, 2, 200, 176
HW = H * W
PC = HW // 128
NSLOT = B * D * HW


CH = 11
NP = CH * 128


def _tc_body(a, s, out_ref):
    t = a[...].T
    for ch in range(CH):
        valid = (s[ch] >= 0).reshape(1, 128)
        sl = slice(ch * 128, (ch + 1) * 128)
        out_ref[0, :, sl] = jnp.where(valid, t[:, sl], 0.0)


@jax.jit
def _tc_transpose(a, sv3):
    return pl.pallas_call(
        _tc_body,
        grid=(B, D, PC // CH),
        in_specs=[
            pl.BlockSpec((NP, 128),
                         lambda b, d, p: (b * 2 * (PC // CH) + d * (PC // CH) + p, 0)),
            pl.BlockSpec((CH, 1, 128),
                         lambda b, d, p: (b * 2 * (PC // CH) + d * (PC // CH) + p, 0, 0)),
        ],
        out_specs=pl.BlockSpec((1, 128, NP),
                               lambda b, d, p: (b, 0, (d * (PC // CH) + p))),
        out_shape=jax.ShapeDtypeStruct((B, 128, D * HW), jnp.float32),
    )(a, sv3)


def kernel(voxel_features, voxel_lin_idx):
    lin = voxel_lin_idx.astype(jnp.int32)
    dense = jnp.zeros((NSLOT, C), dtype=voxel_features.dtype)
    dense = dense.at[lin].set(voxel_features)
    sv3 = jnp.zeros((B * D * PC, 1, 128), dtype=jnp.int32)
    out = _tc_transpose(dense, sv3)
    return out.reshape(B, C * D, H, W)


# speedup vs baseline: 2.4916x; 1.3318x over previous
import functools

import jax
import jax.numpy as jnp
from jax import lax
from jax.experimental import pallas as pl
from jax.experimental.pallas import tpu as pltpu
from jax.experimental.pallas import tpu_sc as plsc

B, C, D, H, W = 4, 128, 2, 200, 176
HW = H * W
PC = HW // 128
NSLOT = B * D * HW
NVOX = 60000

NTEC = 32
SLOTS_PER = NSLOT // NTEC
IDXCHUNK = 6000
NCHUNKS = NVOX // IDXCHUNK
NVREG_I = IDXCHUNK // 16
ST_VREGS = SLOTS_PER // 16
DMACH = (SLOTS_PER + 127) // 128
LISTROWS = DMACH + 1
APAD = NSLOT + 128


def _sc_body(feat_hbm, lin_hbm, a_hbm, sv_hbm,
             idxbuf, st, ids2d, slots2d, rowbuf, sem_g, sem_s):
    cid = lax.axis_index("c")
    sid = lax.axis_index("s")
    wid = sid * 2 + cid
    base = wid * SLOTS_PER
    iota = lax.iota(jnp.int32, 16)

    def init_st(j, carry):
        st[pl.ds(j * 16, 16)] = jnp.full((16,), -1, jnp.int32)
        return carry
    lax.fori_loop(0, ST_VREGS, init_st, 0)

    def chunk_body(ci, carry):
        pltpu.sync_copy(lin_hbm.at[pl.ds(ci * IDXCHUNK, IDXCHUNK)], idxbuf)

        def vreg_body(j, carry2):
            lin = idxbuf[pl.ds(j * 16, 16)]
            m = (lin >= base) & (lin < base + SLOTS_PER)

            @pl.when(jnp.any(m))
            def _():
                loc = lin - base
                ids = ci * IDXCHUNK + j * 16 + iota
                plsc.store_scatter(st, [loc], ids, mask=m)
                npop = plsc.all_reduce_population_count(m)

                @pl.when(jnp.any(npop > 1))
                def _():
                    def cond(m2):
                        return jnp.any(m2)

                    def body(m2):
                        plsc.store_scatter(st, [loc], ids, mask=m2)
                        cur = plsc.load_gather(st, [loc], mask=m)
                        return m & (cur < ids)

                    cur0 = plsc.load_gather(st, [loc], mask=m)
                    lax.while_loop(cond, body, m & (cur0 < ids))
            return carry2

        return lax.fori_loop(0, NVREG_I, vreg_body, carry)

    lax.fori_loop(0, NCHUNKS, chunk_body, 0)

    pltpu.sync_copy(st, sv_hbm.at[pl.ds(base, SLOTS_PER)])

    def comp_body(j, cntvec):
        vals = st[pl.ds(j * 16, 16)]
        m = vals >= 0
        mi = m.astype(jnp.int32)
        pos = cntvec + plsc.cumsum(mi) - mi
        slots = base + j * 16 + iota
        plsc.store_scatter(ids2d, [pos >> 7, pos & 127], vals, mask=m)
        plsc.store_scatter(slots2d, [pos >> 7, pos & 127], slots, mask=m)
        return cntvec + plsc.all_reduce_population_count(m)

    cntvec = lax.fori_loop(0, ST_VREGS, comp_body,
                           jnp.zeros((16,), jnp.int32))

    def pad_body(j, carry):
        offs = cntvec + j * 16 + iota
        plsc.store_scatter(ids2d, [offs >> 7, offs & 127],
                           jnp.zeros((16,), jnp.int32))
        plsc.store_scatter(slots2d, [offs >> 7, offs & 127],
                           NSLOT + (offs - cntvec))
        return carry
    lax.fori_loop(0, 8, pad_body, 0)

    def pump(k, carry):
        @pl.when(jnp.any(cntvec > k * 128))
        def _():
            pltpu.async_copy(feat_hbm.at[ids2d.at[k]], rowbuf, sem_g).wait()
            pltpu.async_copy(rowbuf, a_hbm.at[slots2d.at[k]], sem_s).wait()
        return carry

    lax.fori_loop(0, DMACH, pump, 0)


@jax.jit
def _sc_scatter(feat, lin):
    mesh = plsc.VectorSubcoreMesh(core_axis_name="c", subcore_axis_name="s")
    return pl.kernel(
        _sc_body,
        mesh=mesh,
        compiler_params=pltpu.CompilerParams(needs_layout_passes=False),
        out_type=[
            jax.ShapeDtypeStruct((APAD, C), jnp.float32),
            jax.ShapeDtypeStruct((NSLOT,), jnp.int32),
        ],
        scratch_types=[
            pltpu.VMEM((IDXCHUNK,), jnp.int32),
            pltpu.VMEM((SLOTS_PER,), jnp.int32),
            pltpu.VMEM((LISTROWS, 128), jnp.int32),
            pltpu.VMEM((LISTROWS, 128), jnp.int32),
            pltpu.VMEM((128, C), jnp.float32),
            pltpu.SemaphoreType.DMA,
            pltpu.SemaphoreType.DMA,
        ],
    )(feat, lin)


CH = 11
NP = CH * 128


def _tc_body(a, s, out_ref):
    t = a[...].T
    for ch in range(CH):
        valid = (s[ch] >= 0).reshape(1, 128)
        sl = slice(ch * 128, (ch + 1) * 128)
        out_ref[0, :, sl] = jnp.where(valid, t[:, sl], 0.0)


@jax.jit
def _tc_transpose(a, sv3):
    return pl.pallas_call(
        _tc_body,
        grid=(B, D, PC // CH),
        in_specs=[
            pl.BlockSpec((NP, 128),
                         lambda b, d, p: (b * 2 * (PC // CH) + d * (PC // CH) + p, 0)),
            pl.BlockSpec((CH, 1, 128),
                         lambda b, d, p: (b * 2 * (PC // CH) + d * (PC // CH) + p, 0, 0)),
        ],
        out_specs=pl.BlockSpec((1, 128, NP),
                               lambda b, d, p: (b, 0, (d * (PC // CH) + p))),
        out_shape=jax.ShapeDtypeStruct((B, 128, D * HW), jnp.float32),
    )(a, sv3)


def kernel(voxel_features, voxel_lin_idx):
    lin = voxel_lin_idx.astype(jnp.int32)
    a, sv = _sc_scatter(voxel_features, lin)
    out = _tc_transpose(a, sv.reshape(B * D * PC, 1, 128))
    return out.reshape(B, C * D, H, W)


# speedup vs baseline: 2.5321x; 1.0163x over previous
import functools

import jax
import jax.numpy as jnp
from jax import lax
from jax.experimental import pallas as pl
from jax.experimental.pallas import tpu as pltpu
from jax.experimental.pallas import tpu_sc as plsc

B, C, D, H, W = 4, 128, 2, 200, 176
HW = H * W
PC = HW // 128
NSLOT = B * D * HW
NVOX = 60000

NTEC = 32
SLOTS_PER = NSLOT // NTEC
IDXCHUNK = 6000
NCHUNKS = NVOX // IDXCHUNK
NVREG_I = IDXCHUNK // 16
ST_VREGS = SLOTS_PER // 16
DMACH = (SLOTS_PER + 127) // 128
LISTROWS = DMACH + 1
APAD = NSLOT + 128


def _sc_body(feat_hbm, lin_hbm, a_hbm, sv_hbm,
             ib0, ib1, st, ids2d, slots2d, rb0, rb1,
             sem_i0, sem_i1, sem_g0, sem_g1, sem_s0, sem_s1):
    cid = lax.axis_index("c")
    sid = lax.axis_index("s")
    wid = sid * 2 + cid
    base = wid * SLOTS_PER
    iota = lax.iota(jnp.int32, 16)

    def init_st(j, carry):
        st[pl.ds(j * 16, 16)] = jnp.full((16,), -1, jnp.int32)
        return carry
    lax.fori_loop(0, ST_VREGS, init_st, 0)

    def scan_chunk(ci, idxbuf):
        def vreg_body(j, carry2):
            lin = idxbuf[pl.ds(j * 16, 16)]
            m = (lin >= base) & (lin < base + SLOTS_PER)

            @pl.when(jnp.any(m))
            def _():
                loc = lin - base
                ids = ci * IDXCHUNK + j * 16 + iota
                plsc.store_scatter(st, [loc], ids, mask=m)
                npop = plsc.all_reduce_population_count(m)

                @pl.when(jnp.any(npop > 1))
                def _():
                    def cond(m2):
                        return jnp.any(m2)

                    def body(m2):
                        plsc.store_scatter(st, [loc], ids, mask=m2)
                        cur = plsc.load_gather(st, [loc], mask=m)
                        return m & (cur < ids)

                    cur0 = plsc.load_gather(st, [loc], mask=m)
                    lax.while_loop(cond, body, m & (cur0 < ids))
            return carry2

        lax.fori_loop(0, NVREG_I, vreg_body, 0)

    def stage(ci, ib, sem):
        return pltpu.make_async_copy(
            lin_hbm.at[pl.ds(ci * IDXCHUNK, IDXCHUNK)], ib, sem)

    stage(0, ib0, sem_i0).start()
    for half in range(NCHUNKS // 2):
        k0, k1 = 2 * half, 2 * half + 1
        stage(k0, ib0, sem_i0).wait()
        stage(k1, ib1, sem_i1).start()
        scan_chunk(k0, ib0)
        stage(k1, ib1, sem_i1).wait()
        if k1 + 1 < NCHUNKS:
            stage(k1 + 1, ib0, sem_i0).start()
        scan_chunk(k1, ib1)

    pltpu.sync_copy(st, sv_hbm.at[pl.ds(base, SLOTS_PER)])

    def comp_body(j, cntvec):
        vals = st[pl.ds(j * 16, 16)]
        m = vals >= 0
        mi = m.astype(jnp.int32)
        pos = cntvec + plsc.cumsum(mi) - mi
        slots = base + j * 16 + iota
        plsc.store_scatter(ids2d, [pos >> 7, pos & 127], vals, mask=m)
        plsc.store_scatter(slots2d, [pos >> 7, pos & 127], slots, mask=m)
        return cntvec + plsc.all_reduce_population_count(m)

    cntvec = lax.fori_loop(0, ST_VREGS, comp_body,
                           jnp.zeros((16,), jnp.int32))

    def pad_body(j, carry):
        offs = cntvec + j * 16 + iota
        plsc.store_scatter(ids2d, [offs >> 7, offs & 127],
                           jnp.zeros((16,), jnp.int32))
        plsc.store_scatter(slots2d, [offs >> 7, offs & 127],
                           NSLOT + (offs - cntvec))
        return carry
    lax.fori_loop(0, 8, pad_body, 0)

    def gat(k, rb, sem):
        return pltpu.make_async_copy(feat_hbm.at[ids2d.at[k]], rb, sem)

    def sca(k, rb, sem):
        return pltpu.make_async_copy(rb, a_hbm.at[slots2d.at[k]], sem)

    def active(k):
        return jnp.any(cntvec > k * 128)

    def pump(kk, carry):
        k0 = kk * 2
        k1 = k0 + 1

        @pl.when(active(k0))
        def _():
            @pl.when(kk > 0)
            def _():
                sca(k0, rb0, sem_s0).wait()
            gat(k0, rb0, sem_g0).start()

        @pl.when(active(k1))
        def _():
            @pl.when(kk > 0)
            def _():
                sca(k1, rb1, sem_s1).wait()
            gat(k1, rb1, sem_g1).start()

        @pl.when(active(k0))
        def _():
            gat(k0, rb0, sem_g0).wait()
            sca(k0, rb0, sem_s0).start()

        @pl.when(active(k1))
        def _():
            gat(k1, rb1, sem_g1).wait()
            sca(k1, rb1, sem_s1).start()

        return carry

    lax.fori_loop(0, (DMACH + 1) // 2, pump, 0)

    @pl.when(active(0))
    def _():
        sca(0, rb0, sem_s0).wait()

    @pl.when(active(1))
    def _():
        sca(1, rb1, sem_s1).wait()


@jax.jit
def _sc_scatter(feat, lin):
    mesh = plsc.VectorSubcoreMesh(core_axis_name="c", subcore_axis_name="s")
    return pl.kernel(
        _sc_body,
        mesh=mesh,
        compiler_params=pltpu.CompilerParams(needs_layout_passes=False),
        out_type=[
            jax.ShapeDtypeStruct((APAD, C), jnp.float32),
            jax.ShapeDtypeStruct((NSLOT,), jnp.int32),
        ],
        scratch_types=[
            pltpu.VMEM((IDXCHUNK,), jnp.int32),
            pltpu.VMEM((IDXCHUNK,), jnp.int32),
            pltpu.VMEM((SLOTS_PER,), jnp.int32),
            pltpu.VMEM((LISTROWS, 128), jnp.int32),
            pltpu.VMEM((LISTROWS, 128), jnp.int32),
            pltpu.VMEM((128, C), jnp.float32),
            pltpu.VMEM((128, C), jnp.float32),
            pltpu.SemaphoreType.DMA,
            pltpu.SemaphoreType.DMA,
            pltpu.SemaphoreType.DMA,
            pltpu.SemaphoreType.DMA,
            pltpu.SemaphoreType.DMA,
            pltpu.SemaphoreType.DMA,
        ],
    )(feat, lin)


CH = 11
NP = CH * 128


def _tc_body(a, s, out_ref):
    t = a[...].T
    for ch in range(CH):
        valid = (s[ch] >= 0).reshape(1, 128)
        sl = slice(ch * 128, (ch + 1) * 128)
        out_ref[0, :, sl] = jnp.where(valid, t[:, sl], 0.0)


@jax.jit
def _tc_transpose(a, sv3):
    return pl.pallas_call(
        _tc_body,
        grid=(B, D, PC // CH),
        in_specs=[
            pl.BlockSpec((NP, 128),
                         lambda b, d, p: (b * 2 * (PC // CH) + d * (PC // CH) + p, 0)),
            pl.BlockSpec((CH, 1, 128),
                         lambda b, d, p: (b * 2 * (PC // CH) + d * (PC // CH) + p, 0, 0)),
        ],
        out_specs=pl.BlockSpec((1, 128, NP),
                               lambda b, d, p: (b, 0, (d * (PC // CH) + p))),
        out_shape=jax.ShapeDtypeStruct((B, 128, D * HW), jnp.float32),
    )(a, sv3)


def kernel(voxel_features, voxel_lin_idx):
    lin = voxel_lin_idx.astype(jnp.int32)
    a, sv = _sc_scatter(voxel_features, lin)
    out = _tc_transpose(a, sv.reshape(B * D * PC, 1, 128))
    return out.reshape(B, C * D, H, W)


# speedup vs baseline: 3.1757x; 1.2542x over previous
import functools

import jax
import jax.numpy as jnp
from jax import lax
from jax.experimental import pallas as pl
from jax.experimental.pallas import tpu as pltpu
from jax.experimental.pallas import tpu_sc as plsc

B, C, D, H, W = 4, 128, 2, 200, 176
HW = H * W
PC = HW // 128
NSLOT = B * D * HW
NVOX = 60000

NTEC = 32
SLOTS_PER = NSLOT // NTEC
IDXCHUNK = 6000
NCHUNKS = NVOX // IDXCHUNK
NVREG_I = IDXCHUNK // 16
ST_VREGS = SLOTS_PER // 16
DMACH = (SLOTS_PER + 127) // 128
LISTROWS = DMACH + 1
APAD = NSLOT + 128


def _sc_body(feat_hbm, lin_hbm, a_hbm, sv_hbm,
             ib0, ib1, st, ids2d, slots2d, rb0, rb1,
             sem_i0, sem_i1, sem_g0, sem_g1, sem_s0, sem_s1):
    cid = lax.axis_index("c")
    sid = lax.axis_index("s")
    wid = sid * 2 + cid
    base = wid * SLOTS_PER
    iota = lax.iota(jnp.int32, 16)

    def init_st(j, carry):
        st[pl.ds(j * 16, 16)] = jnp.full((16,), -1, jnp.int32)
        return carry
    lax.fori_loop(0, ST_VREGS, init_st, 0)

    def scan_chunk(ci, idxbuf):
        def vreg_body(j, carry2):
            lin = idxbuf[pl.ds(j * 16, 16)]
            m = (lin >= base) & (lin < base + SLOTS_PER)
            loc = lin - base
            ids = ci * IDXCHUNK + j * 16 + iota
            _, lastm = plsc.scan_count(loc, m)
            plsc.store_scatter(st, [loc], ids, mask=lastm)
            return carry2

        lax.fori_loop(0, NVREG_I, vreg_body, 0)

    def stage(ci, ib, sem):
        return pltpu.make_async_copy(
            lin_hbm.at[pl.ds(ci * IDXCHUNK, IDXCHUNK)], ib, sem)

    stage(0, ib0, sem_i0).start()
    for half in range(NCHUNKS // 2):
        k0, k1 = 2 * half, 2 * half + 1
        stage(k0, ib0, sem_i0).wait()
        stage(k1, ib1, sem_i1).start()
        scan_chunk(k0, ib0)
        stage(k1, ib1, sem_i1).wait()
        if k1 + 1 < NCHUNKS:
            stage(k1 + 1, ib0, sem_i0).start()
        scan_chunk(k1, ib1)

    pltpu.sync_copy(st, sv_hbm.at[pl.ds(base, SLOTS_PER)])

    def comp_body(j, cntvec):
        vals = st[pl.ds(j * 16, 16)]
        m = vals >= 0
        mi = m.astype(jnp.int32)
        pos = cntvec + plsc.cumsum(mi) - mi
        slots = base + j * 16 + iota
        plsc.store_scatter(ids2d, [pos >> 7, pos & 127], vals, mask=m)
        plsc.store_scatter(slots2d, [pos >> 7, pos & 127], slots, mask=m)
        return cntvec + plsc.all_reduce_population_count(m)

    cntvec = lax.fori_loop(0, ST_VREGS, comp_body,
                           jnp.zeros((16,), jnp.int32))

    def pad_body(j, carry):
        offs = cntvec + j * 16 + iota
        plsc.store_scatter(ids2d, [offs >> 7, offs & 127],
                           jnp.zeros((16,), jnp.int32))
        plsc.store_scatter(slots2d, [offs >> 7, offs & 127],
                           NSLOT + (offs - cntvec))
        return carry
    lax.fori_loop(0, 8, pad_body, 0)

    def gat(k, rb, sem):
        return pltpu.make_async_copy(feat_hbm.at[ids2d.at[k]], rb, sem)

    def sca(k, rb, sem):
        return pltpu.make_async_copy(rb, a_hbm.at[slots2d.at[k]], sem)

    def active(k):
        return jnp.any(cntvec > k * 128)

    def pump(kk, carry):
        k0 = kk * 2
        k1 = k0 + 1

        @pl.when(active(k0))
        def _():
            @pl.when(kk > 0)
            def _():
                sca(k0, rb0, sem_s0).wait()
            gat(k0, rb0, sem_g0).start()

        @pl.when(active(k1))
        def _():
            @pl.when(kk > 0)
            def _():
                sca(k1, rb1, sem_s1).wait()
            gat(k1, rb1, sem_g1).start()

        @pl.when(active(k0))
        def _():
            gat(k0, rb0, sem_g0).wait()
            sca(k0, rb0, sem_s0).start()

        @pl.when(active(k1))
        def _():
            gat(k1, rb1, sem_g1).wait()
            sca(k1, rb1, sem_s1).start()

        return carry

    lax.fori_loop(0, (DMACH + 1) // 2, pump, 0)

    @pl.when(active(0))
    def _():
        sca(0, rb0, sem_s0).wait()

    @pl.when(active(1))
    def _():
        sca(1, rb1, sem_s1).wait()


@jax.jit
def _sc_scatter(feat, lin):
    mesh = plsc.VectorSubcoreMesh(core_axis_name="c", subcore_axis_name="s")
    return pl.kernel(
        _sc_body,
        mesh=mesh,
        compiler_params=pltpu.CompilerParams(needs_layout_passes=False),
        out_type=[
            jax.ShapeDtypeStruct((APAD, C), jnp.float32),
            jax.ShapeDtypeStruct((NSLOT,), jnp.int32),
        ],
        scratch_types=[
            pltpu.VMEM((IDXCHUNK,), jnp.int32),
            pltpu.VMEM((IDXCHUNK,), jnp.int32),
            pltpu.VMEM((SLOTS_PER,), jnp.int32),
            pltpu.VMEM((LISTROWS, 128), jnp.int32),
            pltpu.VMEM((LISTROWS, 128), jnp.int32),
            pltpu.VMEM((128, C), jnp.float32),
            pltpu.VMEM((128, C), jnp.float32),
            pltpu.SemaphoreType.DMA,
            pltpu.SemaphoreType.DMA,
            pltpu.SemaphoreType.DMA,
            pltpu.SemaphoreType.DMA,
            pltpu.SemaphoreType.DMA,
            pltpu.SemaphoreType.DMA,
        ],
    )(feat, lin)


CH = 11
NP = CH * 128


def _tc_body(a, s, out_ref):
    t = a[...].T
    for ch in range(CH):
        valid = (s[ch] >= 0).reshape(1, 128)
        sl = slice(ch * 128, (ch + 1) * 128)
        out_ref[0, :, sl] = jnp.where(valid, t[:, sl], 0.0)


@jax.jit
def _tc_transpose(a, sv3):
    return pl.pallas_call(
        _tc_body,
        grid=(B, D, PC // CH),
        in_specs=[
            pl.BlockSpec((NP, 128),
                         lambda b, d, p: (b * 2 * (PC // CH) + d * (PC // CH) + p, 0)),
            pl.BlockSpec((CH, 1, 128),
                         lambda b, d, p: (b * 2 * (PC // CH) + d * (PC // CH) + p, 0, 0)),
        ],
        out_specs=pl.BlockSpec((1, 128, NP),
                               lambda b, d, p: (b, 0, (d * (PC // CH) + p))),
        out_shape=jax.ShapeDtypeStruct((B, 128, D * HW), jnp.float32),
    )(a, sv3)


def kernel(voxel_features, voxel_lin_idx):
    lin = voxel_lin_idx.astype(jnp.int32)
    a, sv = _sc_scatter(voxel_features, lin)
    out = _tc_transpose(a, sv.reshape(B * D * PC, 1, 128))
    return out.reshape(B, C * D, H, W)


# speedup vs baseline: 6.4847x; 2.0420x over previous
import functools

import jax
import jax.numpy as jnp
from jax import lax
from jax.experimental import pallas as pl
from jax.experimental.pallas import tpu as pltpu
from jax.experimental.pallas import tpu_sc as plsc

B, C, D, H, W = 4, 128, 2, 200, 176
HW = H * W
PC = HW // 128
NSLOT = B * D * HW
NVOX = 60000

NTEC = 32
SLOTS_PER = NSLOT // NTEC
IDXCHUNK = 6000
NCHUNKS = NVOX // IDXCHUNK
NVREG_I = IDXCHUNK // 16
ST_VREGS = SLOTS_PER // 16
DMACH = (SLOTS_PER + 127) // 128
LISTROWS = DMACH + 1
APAD = NSLOT + 128


def _sc_body(feat_hbm, lin_hbm, a_hbm, sv_hbm,
             ib0, ib1, st, ids2d, slots2d, rb0, rb1,
             sem_i0, sem_i1, sem_g0, sem_g1, sem_s0, sem_s1):
    cid = lax.axis_index("c")
    sid = lax.axis_index("s")
    wid = sid * 2 + cid
    base = wid * SLOTS_PER
    iota = lax.iota(jnp.int32, 16)

    def init_st(j, carry):
        st[pl.ds(j * 16, 16)] = jnp.full((16,), -1, jnp.int32)
        return carry
    lax.fori_loop(0, ST_VREGS, init_st, 0)

    def scan_chunk(ci, idxbuf):
        def vreg_body(j, carry2):
            lin = idxbuf[pl.ds(j * 16, 16)]
            m = (lin >= base) & (lin < base + SLOTS_PER)
            loc = lin - base
            ids = ci * IDXCHUNK + j * 16 + iota
            _, lastm = plsc.scan_count(loc, m)
            plsc.store_scatter(st, [loc], ids, mask=lastm)
            return carry2

        lax.fori_loop(0, NVREG_I, vreg_body, 0)

    def stage(ci, ib, sem):
        return pltpu.make_async_copy(
            lin_hbm.at[pl.ds(ci * IDXCHUNK, IDXCHUNK)], ib, sem)

    stage(0, ib0, sem_i0).start()
    for half in range(NCHUNKS // 2):
        k0, k1 = 2 * half, 2 * half + 1
        stage(k0, ib0, sem_i0).wait()
        stage(k1, ib1, sem_i1).start()
        scan_chunk(k0, ib0)
        stage(k1, ib1, sem_i1).wait()
        if k1 + 1 < NCHUNKS:
            stage(k1 + 1, ib0, sem_i0).start()
        scan_chunk(k1, ib1)

    pltpu.sync_copy(st, sv_hbm.at[pl.ds(base, SLOTS_PER)])

    def comp_body(j, cntvec):
        vals = st[pl.ds(j * 16, 16)]
        m = vals >= 0
        mi = m.astype(jnp.int32)
        pos = cntvec + plsc.cumsum(mi) - mi
        slots = base + j * 16 + iota
        plsc.store_scatter(ids2d, [pos >> 7, pos & 127], vals, mask=m)
        plsc.store_scatter(slots2d, [pos >> 7, pos & 127], slots, mask=m)
        return cntvec + plsc.all_reduce_population_count(m)

    cntvec = lax.fori_loop(0, ST_VREGS, comp_body,
                           jnp.zeros((16,), jnp.int32))

    def pad_body(j, carry):
        offs = cntvec + j * 16 + iota
        plsc.store_scatter(ids2d, [offs >> 7, offs & 127],
                           jnp.zeros((16,), jnp.int32))
        plsc.store_scatter(slots2d, [offs >> 7, offs & 127],
                           NSLOT + (offs - cntvec))
        return carry
    lax.fori_loop(0, 8, pad_body, 0)

    def gat(k, rb, sem):
        return pltpu.make_async_copy(feat_hbm.at[ids2d.at[k]], rb, sem)

    def sca(k, rb, sem):
        return pltpu.make_async_copy(rb, a_hbm.at[slots2d.at[k]], sem)

    def active(k):
        return jnp.any(cntvec > k * 128)

    def pump(kk, carry):
        k0 = kk * 2
        k1 = k0 + 1

        @pl.when(active(k0))
        def _():
            @pl.when(kk > 0)
            def _():
                sca(k0, rb0, sem_s0).wait()
            gat(k0, rb0, sem_g0).start()

        @pl.when(active(k1))
        def _():
            @pl.when(kk > 0)
            def _():
                sca(k1, rb1, sem_s1).wait()
            gat(k1, rb1, sem_g1).start()

        @pl.when(active(k0))
        def _():
            gat(k0, rb0, sem_g0).wait()
            sca(k0, rb0, sem_s0).start()

        @pl.when(active(k1))
        def _():
            gat(k1, rb1, sem_g1).wait()
            sca(k1, rb1, sem_s1).start()

        return carry

    lax.fori_loop(0, (DMACH + 1) // 2, pump, 0)

    @pl.when(active(0))
    def _():
        sca(0, rb0, sem_s0).wait()

    @pl.when(active(1))
    def _():
        sca(1, rb1, sem_s1).wait()


@jax.jit
def _sc_scatter(feat, lin):
    mesh = plsc.VectorSubcoreMesh(core_axis_name="c", subcore_axis_name="s")
    return pl.kernel(
        _sc_body,
        mesh=mesh,
        compiler_params=pltpu.CompilerParams(needs_layout_passes=False),
        out_type=[
            jax.ShapeDtypeStruct((APAD, C), jnp.float32),
            jax.ShapeDtypeStruct((NSLOT,), jnp.int32),
        ],
        scratch_types=[
            pltpu.VMEM((IDXCHUNK,), jnp.int32),
            pltpu.VMEM((IDXCHUNK,), jnp.int32),
            pltpu.VMEM((SLOTS_PER,), jnp.int32),
            pltpu.VMEM((LISTROWS, 128), jnp.int32),
            pltpu.VMEM((LISTROWS, 128), jnp.int32),
            pltpu.VMEM((128, C), jnp.float32),
            pltpu.VMEM((128, C), jnp.float32),
            pltpu.SemaphoreType.DMA,
            pltpu.SemaphoreType.DMA,
            pltpu.SemaphoreType.DMA,
            pltpu.SemaphoreType.DMA,
            pltpu.SemaphoreType.DMA,
            pltpu.SemaphoreType.DMA,
        ],
    )(feat, lin)


CH = 11
NP = CH * 128
NPC = PC // CH


def _tc_body(a0, a1, s0, s1, out_ref):
    s0t = s0[:, 0, :].T
    s1t = s1[:, 0, :].T
    r = lax.broadcasted_iota(jnp.int32, (2 * C, 2 * C), 0)
    j = lax.broadcasted_iota(jnp.int32, (2 * C, 2 * C), 1)
    perm = (j == 2 * (r % C) + (r // C)).astype(jnp.float32)
    for ch in range(CH):
        sl = slice(ch * 128, (ch + 1) * 128)
        rows0 = a0[sl, :]
        rows1 = a1[sl, :]
        m0 = s0t[:, ch:ch + 1] >= 0
        m1 = s1t[:, ch:ch + 1] >= 0
        g0 = jnp.where(m0 & jnp.isfinite(rows0), rows0, 0.0)
        g1 = jnp.where(m1 & jnp.isfinite(rows1), rows1, 0.0)
        cat = jnp.concatenate([g0, g1], axis=1)
        outr = lax.dot_general(cat, perm, (((1,), (0,)), ((), ())),
                               preferred_element_type=jnp.float32)
        out_ref[0, sl, :] = outr


@jax.jit
def _tc_pack(a, sv3):
    return pl.pallas_call(
        _tc_body,
        grid=(B, NPC),
        in_specs=[
            pl.BlockSpec((NP, 128), lambda b, p: (b * 2 * NPC + p, 0)),
            pl.BlockSpec((NP, 128), lambda b, p: (b * 2 * NPC + NPC + p, 0)),
            pl.BlockSpec((CH, 1, 128), lambda b, p: (b * 2 * NPC + p, 0, 0)),
            pl.BlockSpec((CH, 1, 128),
                         lambda b, p: (b * 2 * NPC + NPC + p, 0, 0)),
        ],
        out_specs=pl.BlockSpec((1, NP, 2 * C), lambda b, p: (b, p, 0)),
        out_shape=jax.ShapeDtypeStruct((B, HW, 2 * C), jnp.float32),
    )(a, a, sv3, sv3)


def kernel(voxel_features, voxel_lin_idx):
    lin = voxel_lin_idx.astype(jnp.int32)
    a, sv = _sc_scatter(voxel_features, lin)
    out = _tc_pack(a, sv.reshape(B * D * PC, 1, 128))
    return out.reshape(B, H, W, C * D).transpose(0, 3, 1, 2)
